# baseline (speedup 1.0000x reference)
import functools

import jax
import jax.numpy as jnp
from jax import lax
from jax.experimental import pallas as pl
from jax.experimental.pallas import tpu as pltpu
from jax.experimental.pallas import tpu_sc as plsc

VOCAB = 100000
D = 16
B = 16384

_info = plsc.get_sparse_core_info()
NC, NS, L = _info.num_cores, _info.num_subcores, _info.num_lanes
NW = NC * NS
BPW = B // NW

_mesh = plsc.VectorSubcoreMesh(core_axis_name="c", subcore_axis_name="s")


@functools.partial(
    pl.kernel,
    out_type=jax.ShapeDtypeStruct((B,), jnp.float32),
    mesh=_mesh,
    scratch_types=[
        pltpu.VMEM((BPW,), jnp.int32),
        pltpu.VMEM((BPW,), jnp.int32),
        pltpu.VMEM((BPW, D), jnp.float32),
        pltpu.VMEM((BPW, D), jnp.float32),
        pltpu.VMEM((BPW,), jnp.float32),
        pltpu.SemaphoreType.DMA,
        pltpu.SemaphoreType.DMA,
    ],
    compiler_params=pltpu.CompilerParams(
        needs_layout_passes=False, use_tc_tiling_on_sc=False),
)
def _w2v_kernel(table_hbm, cen_hbm, ctx_hbm, out_hbm,
                cen_idx, ctx_idx, cen_rows, ctx_rows, sims, sem0, sem1):
    wid = lax.axis_index("s") * NC + lax.axis_index("c")
    base = wid * BPW
    pltpu.sync_copy(cen_hbm.at[pl.ds(base, BPW)], cen_idx)
    pltpu.sync_copy(ctx_hbm.at[pl.ds(base, BPW)], ctx_idx)
    c0 = pltpu.async_copy(table_hbm.at[cen_idx], cen_rows, sem0)
    c1 = pltpu.async_copy(table_hbm.at[ctx_idx], ctx_rows, sem1)
    c0.wait()
    c1.wait()

    lane = lax.iota(jnp.int32, L)

    last = jnp.full((L,), D - 1, jnp.int32)
    masks = [lane == k for k in range(L)]

    def body(g, carry):
        acc = jnp.zeros((L,), jnp.float32)
        for k in range(L):
            j = g * L + k
            prod = cen_rows[j, :] * ctx_rows[j, :]
            s = jnp.cumsum(prod)
            tot = jnp.take_along_axis(s, last, axis=0)
            acc = jnp.where(masks[k], tot, acc)
        sims[pl.ds(g * L, L)] = acc
        return carry

    lax.fori_loop(0, BPW // L, body, None)
    pltpu.sync_copy(sims, out_hbm.at[pl.ds(base, BPW)])


def kernel(pair, label, table):
    del label
    pair = jnp.reshape(pair, (-1, 2)).astype(jnp.int32)
    cen = pair[:, 0]
    ctx = pair[:, 1]
    sims = _w2v_kernel(table, cen, ctx)
    return jnp.reshape(sims, (B, 1))


# speedup vs baseline: 1.5253x; 1.5253x over previous
import functools

import jax
import jax.numpy as jnp
from jax import lax
from jax.experimental import pallas as pl
from jax.experimental.pallas import tpu as pltpu
from jax.experimental.pallas import tpu_sc as plsc

VOCAB = 100000
D = 16
B = 16384

_info = plsc.get_sparse_core_info()
NC, NS, L = _info.num_cores, _info.num_subcores, _info.num_lanes
HALF = B // NC
CHUNK = 2048
NCH = HALF // CHUNK
RED = HALF // NS

_mesh = plsc.VectorSubcoreMesh(core_axis_name="c", subcore_axis_name="s")


@functools.partial(
    pl.kernel,
    out_type=jax.ShapeDtypeStruct((B,), jnp.float32),
    mesh=_mesh,
    scratch_types=[
        pltpu.VMEM_SHARED((NS, HALF), jnp.float32),
        pltpu.VMEM((VOCAB,), jnp.float32),
        pltpu.VMEM((CHUNK,), jnp.int32),
        pltpu.VMEM((CHUNK,), jnp.int32),
        pltpu.VMEM((CHUNK,), jnp.float32),
        pltpu.VMEM((NS, RED), jnp.float32),
        pltpu.VMEM((RED,), jnp.float32),
    ],
    compiler_params=pltpu.CompilerParams(
        needs_layout_passes=False, use_tc_tiling_on_sc=False),
)
def _w2v_kernel(tableT_hbm, cen_hbm, ctx_hbm, out_hbm,
                shared, slice_v, cen_ch, ctx_ch, prod_ch, red_buf, acc):
    d = lax.axis_index("s")
    c = lax.axis_index("c")
    hbase = c * HALF

    pltpu.sync_copy(tableT_hbm.at[d], slice_v)

    def chunk_body(ch, carry):
        base = hbase + ch * CHUNK
        pltpu.sync_copy(cen_hbm.at[pl.ds(base, CHUNK)], cen_ch)
        pltpu.sync_copy(ctx_hbm.at[pl.ds(base, CHUNK)], ctx_ch)

        def group_body(g, carry2):
            ci = cen_ch[pl.ds(g * L, L)]
            xi = ctx_ch[pl.ds(g * L, L)]
            cv = plsc.load_gather(slice_v, [ci])
            xv = plsc.load_gather(slice_v, [xi])
            prod_ch[pl.ds(g * L, L)] = cv * xv
            return carry2

        lax.fori_loop(0, CHUNK // L, group_body, None)
        pltpu.sync_copy(prod_ch, shared.at[d, pl.ds(ch * CHUNK, CHUNK)])
        return carry

    lax.fori_loop(0, NCH, chunk_body, None)
    plsc.subcore_barrier()

    pltpu.sync_copy(shared.at[:, pl.ds(d * RED, RED)], red_buf)

    def red_body(g, carry):
        s = jnp.zeros((L,), jnp.float32)
        for dd in range(NS):
            s = s + red_buf[dd, pl.ds(g * L, L)]
        acc[pl.ds(g * L, L)] = s
        return carry

    lax.fori_loop(0, RED // L, red_body, None)
    pltpu.sync_copy(acc, out_hbm.at[pl.ds(hbase + d * RED, RED)])


def kernel(pair, label, table):
    del label
    pair = jnp.reshape(pair, (-1, 2)).astype(jnp.int32)
    cen = pair[:, 0]
    ctx = pair[:, 1]
    sims = _w2v_kernel(table.T, cen, ctx)
    return jnp.reshape(sims, (B, 1))
